# Initial kernel scaffold; baseline (speedup 1.0000x reference)
import jax
import jax.numpy as jnp
from jax.experimental import pallas as pl


def kernel(x, permutations, weight, bias):
    raise NotImplementedError("write your pallas kernel here")


# speedup vs baseline: 8.6076x; 8.6076x over previous
import jax
import jax.numpy as jnp
from jax.experimental import pallas as pl

_BO = 512


def _fused_kernel(perm_ref, w_ref, bias_ref, x_ref, out_ref):
    k = x_ref.shape[1]
    bo = out_ref.shape[1]
    perm = perm_ref[...]
    wv = w_ref[...]
    row = jax.lax.broadcasted_iota(jnp.int32, (k, bo), 0)
    acc = jnp.zeros((k, bo), jnp.float32)
    for s in range(perm.shape[0]):
        acc = acc + jnp.where(row == perm[s : s + 1, :], wv[s : s + 1, :], 0.0)
    out_ref[...] = (
        jnp.dot(x_ref[...], acc, preferred_element_type=jnp.float32)
        + bias_ref[...]
    )


def kernel(x, permutations, weight, bias):
    lead = x.shape[:-1]
    k = x.shape[-1]
    t = 1
    for d in lead:
        t *= d
    x2 = x.reshape(t, k)
    o, s = permutations.shape
    perm_t = permutations.T
    w_t = weight.T
    bias2 = bias.reshape(1, o)
    nj = o // _BO
    out = pl.pallas_call(
        _fused_kernel,
        grid=(nj,),
        in_specs=[
            pl.BlockSpec((s, _BO), lambda j: (0, j)),
            pl.BlockSpec((s, _BO), lambda j: (0, j)),
            pl.BlockSpec((1, _BO), lambda j: (0, j)),
            pl.BlockSpec((t, k), lambda j: (0, 0)),
        ],
        out_specs=pl.BlockSpec((t, _BO), lambda j: (0, j)),
        out_shape=jax.ShapeDtypeStruct((t, o), jnp.float32),
    )(perm_t, w_t, bias2, x2)
    return out.reshape(*lead, o)


# speedup vs baseline: 10.1157x; 1.1752x over previous
import jax
import jax.numpy as jnp
from jax.experimental import pallas as pl

_BO = 512


def _fused_kernel(perm_ref, w_ref, bias_ref, x_ref, out_ref):
    k = x_ref.shape[1]
    bo = out_ref.shape[1]
    perm = perm_ref[...]
    wv = w_ref[...].astype(jnp.bfloat16)
    row = jax.lax.broadcasted_iota(jnp.int16, (k, bo), 0)
    acc = jnp.zeros((k, bo), jnp.bfloat16)
    for s in range(perm.shape[0]):
        acc = acc + jnp.where(
            row == perm[s : s + 1, :], wv[s : s + 1, :], jnp.bfloat16(0.0)
        )
    out_ref[...] = (
        jnp.dot(x_ref[...], acc, preferred_element_type=jnp.float32)
        + bias_ref[...]
    )


def kernel(x, permutations, weight, bias):
    lead = x.shape[:-1]
    k = x.shape[-1]
    t = 1
    for d in lead:
        t *= d
    x2 = x.reshape(t, k).astype(jnp.bfloat16)
    o, s = permutations.shape
    perm_t = permutations.T.astype(jnp.int16)
    w_t = weight.T
    bias2 = bias.reshape(1, o)
    nj = o // _BO
    out = pl.pallas_call(
        _fused_kernel,
        grid=(nj,),
        in_specs=[
            pl.BlockSpec((s, _BO), lambda j: (0, j)),
            pl.BlockSpec((s, _BO), lambda j: (0, j)),
            pl.BlockSpec((1, _BO), lambda j: (0, j)),
            pl.BlockSpec((t, k), lambda j: (0, 0)),
        ],
        out_specs=pl.BlockSpec((t, _BO), lambda j: (0, j)),
        out_shape=jax.ShapeDtypeStruct((t, o), jnp.float32),
    )(perm_t, w_t, bias2, x2)
    return out.reshape(*lead, o)


# speedup vs baseline: 12.0653x; 1.1927x over previous
import jax
import jax.numpy as jnp
from jax.experimental import pallas as pl
from jax.experimental.pallas import tpu as pltpu

_BO = 512


def _fused_kernel(perm_ref, w_ref, bias_ref, x_ref, out_ref, xbf_ref):
    k = x_ref.shape[1]
    bo = out_ref.shape[1]

    @pl.when(pl.program_id(0) == 0)
    def _cast_x():
        xbf_ref[...] = x_ref[...].astype(jnp.bfloat16)

    perm = perm_ref[...]
    wv = w_ref[...].astype(jnp.bfloat16)
    row = jax.lax.broadcasted_iota(jnp.int16, (k, bo), 0)
    acc = jnp.zeros((k, bo), jnp.bfloat16)
    for s in range(perm.shape[0]):
        acc = acc + jnp.where(
            row == perm[s : s + 1, :], wv[s : s + 1, :], jnp.bfloat16(0.0)
        )
    out_ref[...] = (
        jnp.dot(xbf_ref[...], acc, preferred_element_type=jnp.float32)
        + bias_ref[...]
    )


def kernel(x, permutations, weight, bias):
    lead = x.shape[:-1]
    k = x.shape[-1]
    t = 1
    for d in lead:
        t *= d
    x2 = x.reshape(t, k)
    o, s = permutations.shape
    perm_t = permutations.T.astype(jnp.int16)
    w_t = weight.T
    bias2 = bias.reshape(1, o)
    nj = o // _BO
    out = pl.pallas_call(
        _fused_kernel,
        grid=(nj,),
        in_specs=[
            pl.BlockSpec((s, _BO), lambda j: (0, j)),
            pl.BlockSpec((s, _BO), lambda j: (0, j)),
            pl.BlockSpec((1, _BO), lambda j: (0, j)),
            pl.BlockSpec((t, k), lambda j: (0, 0)),
        ],
        out_specs=pl.BlockSpec((t, _BO), lambda j: (0, j)),
        out_shape=jax.ShapeDtypeStruct((t, o), jnp.float32),
        scratch_shapes=[pltpu.VMEM((t, k), jnp.bfloat16)],
    )(perm_t, w_t, bias2, x2)
    return out.reshape(*lead, o)
